# Initial kernel scaffold; baseline (speedup 1.0000x reference)
import jax
import jax.numpy as jnp
from jax.experimental import pallas as pl


def kernel(preds, targets):
    raise NotImplementedError("write your pallas kernel here")


# speedup vs baseline: 22.0190x; 22.0190x over previous
import jax
import jax.numpy as jnp
from jax import lax
from jax.experimental import pallas as pl
from jax.experimental.pallas import tpu as pltpu

_STEP = 10
_POS_THRESH = 0.5


def _fused_body(preds_ref, targets_ref, out_ref):
    p = preds_ref[...]
    t = targets_ref[...]
    losses = jnp.maximum(p, 0.0) - p * t + jnp.log1p(jnp.exp(-jnp.abs(p)))
    masked = jnp.where(t < _POS_THRESH, losses, 0.0)
    bits = lax.bitcast_convert_type(masked, jnp.int32)

    count = jnp.sum((bits > 0).astype(jnp.int32))
    k = (count + (_STEP - 1)) // _STEP
    total = jnp.sum(losses)

    def body(_, carry):
        lo, hi = carry
        mid = lo + (hi - lo) // 2
        c = jnp.sum((bits > mid).astype(jnp.int32))
        big = c >= k
        return (jnp.where(big, mid + 1, lo), jnp.where(big, hi, mid))

    lo0 = jnp.int32(0)
    hi0 = jnp.int32(0x7F800000)
    _, vbits = lax.fori_loop(0, 31, body, (lo0, hi0))

    dropped = jnp.sum(jnp.where(bits >= vbits, masked, 0.0))
    out_ref[0, 0] = (total - dropped) / jnp.float32(p.size)


def kernel(preds, targets):
    out = pl.pallas_call(
        _fused_body,
        out_shape=jax.ShapeDtypeStruct((1, 1), jnp.float32),
        out_specs=pl.BlockSpec(memory_space=pltpu.SMEM),
    )(preds, targets)
    return out[0, 0]
